# Initial kernel scaffold; baseline (speedup 1.0000x reference)
import jax
import jax.numpy as jnp
from jax.experimental import pallas as pl


def kernel(grid, points, W1_0, b1_0, W2_0, b2_0, W1_1, b1_1, W2_1, b2_1, W1_2, b1_2, W2_2, b2_2, chunk_size):
    raise NotImplementedError("write your pallas kernel here")


# speedup vs baseline: 1.2343x; 1.2343x over previous
import functools

import jax
import jax.numpy as jnp
from jax.experimental import pallas as pl

_SCALES = [(0.02, 16), (0.05, 32), (0.1, 64)]
_HID = 128
_OUT = 64
_G = 4096
_N = 16384
_K = 64


def _d2_body(gx_ref, gy_ref, px_ref, py_ref, out_ref):
    gx = gx_ref[...]
    gy = gy_ref[...]
    px = px_ref[...]
    py = py_ref[...]
    dx = gx - px
    dy = gy - py
    out_ref[...] = dx * dx + dy * dy


def _mlp_body(rx_ref, ry_ref, d2_ref, w1_ref, b1_ref, w2_ref, b2_ref, out_ref):
    rx = rx_ref[...]
    ry = ry_ref[...]
    d2 = d2_ref[...]
    dist = jnp.sqrt(jnp.maximum(d2, 1e-24))
    for s, (radius, k) in enumerate(_SCALES):
        w1x = w1_ref[2 * s : 2 * s + 1, :]
        w1y = w1_ref[2 * s + 1 : 2 * s + 2, :]
        b1 = b1_ref[s : s + 1, :]
        w2 = w2_ref[s * _HID : (s + 1) * _HID, :]
        b2 = b2_ref[s : s + 1, :]
        acc = jnp.full(out_ref[:, : _OUT].shape, -jnp.inf, dtype=jnp.float32)
        anyv = jnp.zeros(acc[:, :1].shape, dtype=jnp.bool_)
        for j in range(k):
            valid = dist[:, j : j + 1] < radius
            h = jnp.maximum(rx[:, j : j + 1] * w1x + ry[:, j : j + 1] * w1y + b1, 0.0)
            f = jnp.dot(h, w2, preferred_element_type=jnp.float32)
            acc = jnp.maximum(acc, jnp.where(valid, f, -jnp.inf))
            anyv = jnp.logical_or(anyv, valid)
        pooled = jnp.where(anyv, acc + b2, 0.0)
        out_ref[:, s * _OUT : (s + 1) * _OUT] = pooled


def kernel(grid, points, W1_0, b1_0, W2_0, b2_0, W1_1, b1_1, W2_1, b2_1,
           W1_2, b1_2, W2_2, b2_2, chunk_size):
    gx = grid[:, 0:1]
    gy = grid[:, 1:2]
    px = points[:, 0].reshape(1, _N)
    py = points[:, 1].reshape(1, _N)

    bg = 128
    d2 = pl.pallas_call(
        _d2_body,
        grid=(_G // bg,),
        in_specs=[
            pl.BlockSpec((bg, 1), lambda i: (i, 0)),
            pl.BlockSpec((bg, 1), lambda i: (i, 0)),
            pl.BlockSpec((1, _N), lambda i: (0, 0)),
            pl.BlockSpec((1, _N), lambda i: (0, 0)),
        ],
        out_specs=pl.BlockSpec((bg, _N), lambda i: (i, 0)),
        out_shape=jax.ShapeDtypeStruct((_G, _N), jnp.float32),
    )(gx, gy, px, py)

    neg, idx = jax.lax.top_k(-d2, _K)
    d2top = -neg
    pos = points[idx]
    relx = pos[:, :, 0] - gx
    rely = pos[:, :, 1] - gy

    w1 = jnp.stack([W1_0[:, 0], W1_0[:, 1], W1_1[:, 0], W1_1[:, 1],
                    W1_2[:, 0], W1_2[:, 1]], axis=0)
    b1 = jnp.stack([b1_0, b1_1, b1_2], axis=0)
    w2 = jnp.concatenate([W2_0.T, W2_1.T, W2_2.T], axis=0)
    b2 = jnp.stack([b2_0, b2_1, b2_2], axis=0)

    bm = 256
    out = pl.pallas_call(
        _mlp_body,
        grid=(_G // bm,),
        in_specs=[
            pl.BlockSpec((bm, _K), lambda i: (i, 0)),
            pl.BlockSpec((bm, _K), lambda i: (i, 0)),
            pl.BlockSpec((bm, _K), lambda i: (i, 0)),
            pl.BlockSpec((6, _HID), lambda i: (0, 0)),
            pl.BlockSpec((3, _HID), lambda i: (0, 0)),
            pl.BlockSpec((3 * _HID, _OUT), lambda i: (0, 0)),
            pl.BlockSpec((3, _OUT), lambda i: (0, 0)),
        ],
        out_specs=pl.BlockSpec((bm, 3 * _OUT), lambda i: (i, 0)),
        out_shape=jax.ShapeDtypeStruct((_G, 3 * _OUT), jnp.float32),
    )(relx, rely, d2top, w1, b1, w2, b2)
    return out


# speedup vs baseline: 31.2193x; 25.2924x over previous
import functools

import jax
import jax.numpy as jnp
from jax import lax
from jax.experimental import pallas as pl
from jax.experimental.pallas import tpu as pltpu
from jax.experimental.pallas import tpu_sc as plsc

_SCALES = [(0.02, 16), (0.05, 32), (0.1, 64)]
_HID = 128
_OUT = 64
_G = 4096
_N = 16384
_K = 64
_NGRP = 1024
_CAP = 128


def _gmin_tau_body(gx_ref, gy_ref, px3_ref, py3_ref, gmin_ref, tau_ref):
    gx = gx_ref[...]
    gy = gy_ref[...]
    gmin = jnp.full((gx.shape[0], _NGRP), jnp.inf, dtype=jnp.float32)
    for s in range(16):
        dx = gx - px3_ref[s : s + 1, :]
        dy = gy - py3_ref[s : s + 1, :]
        gmin = jnp.minimum(gmin, dx * dx + dy * dy)
    gmin_ref[...] = gmin

    rmax2 = jnp.float32(0.01)

    def body(_, lohi):
        lo, hi = lohi
        mid = 0.5 * (lo + hi)
        c = jnp.sum((gmin < mid).astype(jnp.float32), axis=1, keepdims=True)
        p = c >= 64.0
        return (jnp.where(p, lo, mid), jnp.where(p, mid, hi))

    lo0 = jnp.zeros_like(gx)
    hi0 = jnp.full_like(gx, rmax2)
    _, hi = lax.fori_loop(0, 16, body, (lo0, hi0))
    tau_ref[...] = hi * jnp.float32(1.0 + 2e-6)


def _ce(a, b):
    m = a[0] <= b[0]
    lo = (jnp.where(m, a[0], b[0]), jnp.where(m, a[1], b[1]))
    hi = (jnp.where(m, b[0], a[0]), jnp.where(m, b[1], a[1]))
    return lo, hi


def _vrev(a):
    return (lax.rev(a[0], (0,)), lax.rev(a[1], (0,)))


def _vsort(a):
    k, v = plsc.sort_key_val(a[0], a[1])
    return (k, v)


def _bmerge32(a, b):
    lo, hi = _ce(a, b)
    return _vsort(lo), _vsort(hi)


def _merge2(a, b):
    return _bmerge32(a, _vrev(b))


def _sort128_low64(vregs):
    v = [_vsort(x) for x in vregs]
    s32 = []
    for i in range(0, 8, 2):
        lo, hi = _merge2(v[i], v[i + 1])
        s32 += [lo, hi]
    s64 = []
    for i in range(0, 8, 4):
        a0, a1, b0, b1 = s32[i], s32[i + 1], s32[i + 2], s32[i + 3]
        w2, w3 = _vrev(b1), _vrev(b0)
        l0, h0 = _ce(a0, w2)
        l1, h1 = _ce(a1, w3)
        ll, lh = _bmerge32(l0, l1)
        hl, hh = _bmerge32(h0, h1)
        s64 += [ll, lh, hl, hh]
    a = s64[0:4]
    b = s64[4:8]
    w = [a[0], a[1], a[2], a[3], _vrev(b[3]), _vrev(b[2]), _vrev(b[1]), _vrev(b[0])]
    l = [_ce(w[i], w[i + 4])[0] for i in range(4)]
    p0, _h0 = _ce(l[0], l[2])
    p1, _h1 = _ce(l[1], l[3])
    q0, q1 = _bmerge32(p0, p1)
    r0, r1 = _bmerge32(_h0, _h1)
    return [q0, q1, r0, r1]


def _sc_body(gmin_hbm, tau_hbm, gx_hbm, gy_hbm, px_hbm, py_hbm,
             od2_hbm, odx_hbm, ody_hbm,
             px_v, py_v, tau_v, gx_v, gy_v, gmin_v, gid_v, cd2_v, cdx_v, cdy_v,
             out_d2_v, out_dx_v, out_dy_v):
    i32 = jnp.int32
    wid = lax.axis_index("s") * 2 + lax.axis_index("c")
    wbase = wid * 128

    pltpu.sync_copy(px_hbm, px_v)
    pltpu.sync_copy(py_hbm, py_v)
    pltpu.sync_copy(tau_hbm.at[pl.ds(wbase, 128)], tau_v)
    pltpu.sync_copy(gx_hbm.at[pl.ds(wbase, 128)], gx_v)
    pltpu.sync_copy(gy_hbm.at[pl.ds(wbase, 128)], gy_v)

    iota = lax.iota(i32, 16)
    inf16 = jnp.full((16,), jnp.inf, dtype=jnp.float32)

    def do_chunk(chunk, _):
        base_l = chunk * 16
        pltpu.sync_copy(gmin_hbm.at[pl.ds((wbase + base_l) * _NGRP, 16 * _NGRP)],
                        gmin_v)

        def do_row(r, _r):
            rl = base_l + r
            tsp = plsc.load_gather(tau_v, [jnp.full((16,), rl, i32)])
            gxs = plsc.load_gather(gx_v, [jnp.full((16,), rl, i32)])
            gys = plsc.load_gather(gy_v, [jnp.full((16,), rl, i32)])

            def p1(jv, gcnt):
                gv = gmin_v[pl.ds(r * _NGRP + jv * 16, 16)]
                m = gv < tsp
                mi = m.astype(i32)
                pos = gcnt + plsc.cumsum(mi) - 1
                plsc.store_scatter(gid_v, [pos], iota + jv * 16, mask=m)
                return gcnt + jnp.sum(mi)

            gcnt = lax.fori_loop(0, _NGRP // 16, p1, i32(0))

            def pinit(w, _w):
                cd2_v[pl.ds(r * _CAP + w * 16, 16)] = inf16
                return 0

            lax.fori_loop(0, _CAP // 16, pinit, 0)

            def p2(b, cnt):
                gvreg = gid_v[pl.ds(b * 16, 16)]
                lanemask = (iota + b * 16) < gcnt
                cc = cnt
                for s in range(16):
                    pidx = gvreg + s * _NGRP
                    pxv = plsc.load_gather(px_v, [pidx], mask=lanemask)
                    pyv = plsc.load_gather(py_v, [pidx], mask=lanemask)
                    dxv = pxv - gxs
                    dyv = pyv - gys
                    d2v = dxv * dxv + dyv * dyv
                    d2v = jnp.where(lanemask, d2v, jnp.inf)
                    cm = (d2v < tsp) & (cc <= _CAP - 16)
                    cmi = cm.astype(i32)
                    pos = r * _CAP + cc + plsc.cumsum(cmi) - 1
                    plsc.store_scatter(cd2_v, [pos], d2v, mask=cm)
                    plsc.store_scatter(cdx_v, [pos], dxv, mask=cm)
                    plsc.store_scatter(cdy_v, [pos], dyv, mask=cm)
                    cc = cc + jnp.sum(cmi)
                return cc

            nv = (gcnt + 15) // 16
            lax.fori_loop(0, nv, p2, i32(0))

            vregs = [(cd2_v[pl.ds(r * _CAP + i * 16, 16)], iota + i * 16)
                     for i in range(_CAP // 16)]
            low = _sort128_low64(vregs)
            for i in range(4):
                sk, sv = low[i]
                dxs = plsc.load_gather(cdx_v, [sv + r * _CAP])
                dys = plsc.load_gather(cdy_v, [sv + r * _CAP])
                out_d2_v[pl.ds(r * _K + i * 16, 16)] = sk
                out_dx_v[pl.ds(r * _K + i * 16, 16)] = dxs
                out_dy_v[pl.ds(r * _K + i * 16, 16)] = dys
            return 0

        lax.fori_loop(0, 16, do_row, 0)

        obase = (wbase + base_l) * _K
        pltpu.sync_copy(out_d2_v, od2_hbm.at[pl.ds(obase, 16 * _K)])
        pltpu.sync_copy(out_dx_v, odx_hbm.at[pl.ds(obase, 16 * _K)])
        pltpu.sync_copy(out_dy_v, ody_hbm.at[pl.ds(obase, 16 * _K)])
        return 0

    lax.fori_loop(0, 8, do_chunk, 0)


def _mlp_body(rx_ref, ry_ref, d2_ref, w1_ref, b1_ref, w2_ref, b2_ref, out_ref):
    rx = rx_ref[...]
    ry = ry_ref[...]
    d2 = d2_ref[...]
    dist = jnp.sqrt(jnp.maximum(d2, 1e-24))
    for s, (radius, k) in enumerate(_SCALES):
        w1x = w1_ref[2 * s : 2 * s + 1, :]
        w1y = w1_ref[2 * s + 1 : 2 * s + 2, :]
        b1 = b1_ref[s : s + 1, :]
        w2 = w2_ref[s * _HID : (s + 1) * _HID, :]
        b2 = b2_ref[s : s + 1, :]
        acc = jnp.full(out_ref[:, : _OUT].shape, -jnp.inf, dtype=jnp.float32)
        anyv = jnp.zeros(acc[:, :1].shape, dtype=jnp.bool_)
        for j in range(k):
            valid = dist[:, j : j + 1] < radius
            h = jnp.maximum(rx[:, j : j + 1] * w1x + ry[:, j : j + 1] * w1y + b1, 0.0)
            f = jnp.dot(h, w2, preferred_element_type=jnp.float32)
            acc = jnp.maximum(acc, jnp.where(valid, f, -jnp.inf))
            anyv = jnp.logical_or(anyv, valid)
        pooled = jnp.where(anyv, acc + b2, 0.0)
        out_ref[:, s * _OUT : (s + 1) * _OUT] = pooled


def kernel(grid, points, W1_0, b1_0, W2_0, b2_0, W1_1, b1_1, W2_1, b2_1,
           W1_2, b1_2, W2_2, b2_2, chunk_size):
    gx = grid[:, 0:1]
    gy = grid[:, 1:2]
    px3 = points[:, 0].reshape(16, _NGRP)
    py3 = points[:, 1].reshape(16, _NGRP)

    bg = 512
    gmin, tau = pl.pallas_call(
        _gmin_tau_body,
        grid=(_G // bg,),
        in_specs=[
            pl.BlockSpec((bg, 1), lambda i: (i, 0)),
            pl.BlockSpec((bg, 1), lambda i: (i, 0)),
            pl.BlockSpec((16, _NGRP), lambda i: (0, 0)),
            pl.BlockSpec((16, _NGRP), lambda i: (0, 0)),
        ],
        out_specs=[
            pl.BlockSpec((bg, _NGRP), lambda i: (i, 0)),
            pl.BlockSpec((bg, 1), lambda i: (i, 0)),
        ],
        out_shape=[
            jax.ShapeDtypeStruct((_G, _NGRP), jnp.float32),
            jax.ShapeDtypeStruct((_G, 1), jnp.float32),
        ],
    )(gx, gy, px3, py3)

    mesh = plsc.VectorSubcoreMesh(core_axis_name="c", subcore_axis_name="s")
    sc = functools.partial(
        pl.kernel,
        mesh=mesh,
        compiler_params=pltpu.CompilerParams(needs_layout_passes=False),
        out_type=[
            jax.ShapeDtypeStruct((_G * _K,), jnp.float32),
            jax.ShapeDtypeStruct((_G * _K,), jnp.float32),
            jax.ShapeDtypeStruct((_G * _K,), jnp.float32),
        ],
        scratch_types=[
            pltpu.VMEM((_N,), jnp.float32),
            pltpu.VMEM((_N,), jnp.float32),
            pltpu.VMEM((128,), jnp.float32),
            pltpu.VMEM((128,), jnp.float32),
            pltpu.VMEM((128,), jnp.float32),
            pltpu.VMEM((16 * _NGRP,), jnp.float32),
            pltpu.VMEM((_CAP + 16,), jnp.int32),
            pltpu.VMEM((16 * _CAP,), jnp.float32),
            pltpu.VMEM((16 * _CAP,), jnp.float32),
            pltpu.VMEM((16 * _CAP,), jnp.float32),
            pltpu.VMEM((16 * _K,), jnp.float32),
            pltpu.VMEM((16 * _K,), jnp.float32),
            pltpu.VMEM((16 * _K,), jnp.float32),
        ],
    )(_sc_body)

    d2s, dxs, dys = sc(
        gmin.reshape(-1), tau.reshape(-1), gx.reshape(-1), gy.reshape(-1),
        points[:, 0], points[:, 1],
    )
    d2s = d2s.reshape(_G, _K)
    dxs = dxs.reshape(_G, _K)
    dys = dys.reshape(_G, _K)

    w1 = jnp.stack([W1_0[:, 0], W1_0[:, 1], W1_1[:, 0], W1_1[:, 1],
                    W1_2[:, 0], W1_2[:, 1]], axis=0)
    b1 = jnp.stack([b1_0, b1_1, b1_2], axis=0)
    w2 = jnp.concatenate([W2_0.T, W2_1.T, W2_2.T], axis=0)
    b2 = jnp.stack([b2_0, b2_1, b2_2], axis=0)

    bm = 256
    out = pl.pallas_call(
        _mlp_body,
        grid=(_G // bm,),
        in_specs=[
            pl.BlockSpec((bm, _K), lambda i: (i, 0)),
            pl.BlockSpec((bm, _K), lambda i: (i, 0)),
            pl.BlockSpec((bm, _K), lambda i: (i, 0)),
            pl.BlockSpec((6, _HID), lambda i: (0, 0)),
            pl.BlockSpec((3, _HID), lambda i: (0, 0)),
            pl.BlockSpec((3 * _HID, _OUT), lambda i: (0, 0)),
            pl.BlockSpec((3, _OUT), lambda i: (0, 0)),
        ],
        out_specs=pl.BlockSpec((bm, 3 * _OUT), lambda i: (i, 0)),
        out_shape=jax.ShapeDtypeStruct((_G, 3 * _OUT), jnp.float32),
    )(dxs, dys, d2s, w1, b1, w2, b2)
    return out


# speedup vs baseline: 33.7111x; 1.0798x over previous
import functools

import jax
import jax.numpy as jnp
from jax import lax
from jax.experimental import pallas as pl
from jax.experimental.pallas import tpu as pltpu
from jax.experimental.pallas import tpu_sc as plsc

_SCALES = [(0.02, 16), (0.05, 32), (0.1, 64)]
_HID = 128
_OUT = 64
_G = 4096
_N = 16384
_K = 64
_NGRP = 1024
_CAP = 128


def _gmin_tau_body(gx_ref, gy_ref, px3_ref, py3_ref, gmin_ref, tau_ref):
    gx = gx_ref[...]
    gy = gy_ref[...]
    gmin = jnp.full((gx.shape[0], _NGRP), jnp.inf, dtype=jnp.float32)
    for s in range(16):
        dx = gx - px3_ref[s : s + 1, :]
        dy = gy - py3_ref[s : s + 1, :]
        gmin = jnp.minimum(gmin, dx * dx + dy * dy)
    gmin_ref[...] = gmin

    rmax2 = jnp.float32(0.01)

    def body(_, st):
        lo, clo, hi, chi = st
        denom = jnp.maximum(chi - clo, 1.0)
        frac = jnp.clip((64.0 - clo) / denom, 0.08, 0.92)
        mid = lo + (hi - lo) * frac
        c = jnp.sum((gmin < mid).astype(jnp.float32), axis=1, keepdims=True)
        p = c >= 64.0
        return (jnp.where(p, lo, mid), jnp.where(p, clo, c),
                jnp.where(p, mid, hi), jnp.where(p, c, chi))

    lo0 = jnp.zeros_like(gx)
    hi0 = jnp.full_like(gx, rmax2)
    chi0 = jnp.sum((gmin < rmax2).astype(jnp.float32), axis=1, keepdims=True)
    _, _, hi, _ = lax.fori_loop(0, 8, body, (lo0, jnp.zeros_like(gx), hi0, chi0))
    tau_ref[...] = hi * jnp.float32(1.0 + 2e-6)


def _ce(a, b):
    m = a[0] <= b[0]
    lo = (jnp.where(m, a[0], b[0]), jnp.where(m, a[1], b[1]))
    hi = (jnp.where(m, b[0], a[0]), jnp.where(m, b[1], a[1]))
    return lo, hi


def _vrev(a):
    return (lax.rev(a[0], (0,)), lax.rev(a[1], (0,)))


def _vsort(a):
    k, v = plsc.sort_key_val(a[0], a[1])
    return (k, v)


def _bmerge32(a, b):
    lo, hi = _ce(a, b)
    return _vsort(lo), _vsort(hi)


def _merge2(a, b):
    return _bmerge32(a, _vrev(b))


def _sort128_low64(vregs):
    v = [_vsort(x) for x in vregs]
    s32 = []
    for i in range(0, 8, 2):
        lo, hi = _merge2(v[i], v[i + 1])
        s32 += [lo, hi]
    s64 = []
    for i in range(0, 8, 4):
        a0, a1, b0, b1 = s32[i], s32[i + 1], s32[i + 2], s32[i + 3]
        w2, w3 = _vrev(b1), _vrev(b0)
        l0, h0 = _ce(a0, w2)
        l1, h1 = _ce(a1, w3)
        ll, lh = _bmerge32(l0, l1)
        hl, hh = _bmerge32(h0, h1)
        s64 += [ll, lh, hl, hh]
    a = s64[0:4]
    b = s64[4:8]
    w = [a[0], a[1], a[2], a[3], _vrev(b[3]), _vrev(b[2]), _vrev(b[1]), _vrev(b[0])]
    l = [_ce(w[i], w[i + 4])[0] for i in range(4)]
    p0, _h0 = _ce(l[0], l[2])
    p1, _h1 = _ce(l[1], l[3])
    q0, q1 = _bmerge32(p0, p1)
    r0, r1 = _bmerge32(_h0, _h1)
    return [q0, q1, r0, r1]


def _sc_body(gmin_hbm, tau_hbm, gx_hbm, gy_hbm, px_hbm, py_hbm,
             od2_hbm, odx_hbm, ody_hbm,
             px_v, py_v, tau_v, gx_v, gy_v, gmin_v, gid_v, cd2_v, cdx_v, cdy_v,
             out_d2_v, out_dx_v, out_dy_v):
    i32 = jnp.int32
    wid = lax.axis_index("s") * 2 + lax.axis_index("c")
    wbase = wid * 128

    pltpu.sync_copy(px_hbm, px_v)
    pltpu.sync_copy(py_hbm, py_v)
    pltpu.sync_copy(tau_hbm.at[pl.ds(wbase, 128)], tau_v)
    pltpu.sync_copy(gx_hbm.at[pl.ds(wbase, 128)], gx_v)
    pltpu.sync_copy(gy_hbm.at[pl.ds(wbase, 128)], gy_v)

    iota = lax.iota(i32, 16)
    inf16 = jnp.full((16,), jnp.inf, dtype=jnp.float32)

    def do_chunk(chunk, _):
        base_l = chunk * 16
        pltpu.sync_copy(gmin_hbm.at[pl.ds((wbase + base_l) * _NGRP, 16 * _NGRP)],
                        gmin_v)

        def do_row(r, _r):
            rl = base_l + r
            tsp = plsc.load_gather(tau_v, [jnp.full((16,), rl, i32)])
            gxs = plsc.load_gather(gx_v, [jnp.full((16,), rl, i32)])
            gys = plsc.load_gather(gy_v, [jnp.full((16,), rl, i32)])

            def p1(jv, gcnt):
                gv = gmin_v[pl.ds(r * _NGRP + jv * 16, 16)]
                m = gv < tsp
                mi = m.astype(i32)
                pos = gcnt + plsc.cumsum(mi) - 1
                plsc.store_scatter(gid_v, [pos], iota + jv * 16, mask=m)
                return gcnt + jnp.sum(mi)

            gcnt = lax.fori_loop(0, _NGRP // 16, p1, i32(0))

            def pinit(w, _w):
                cd2_v[pl.ds(r * _CAP + w * 16, 16)] = inf16
                return 0

            lax.fori_loop(0, _CAP // 16, pinit, 0)

            def p2(b, cnt):
                gvreg = gid_v[pl.ds(b * 16, 16)]
                lanemask = (iota + b * 16) < gcnt
                cc = cnt
                for s in range(16):
                    pidx = gvreg + s * _NGRP
                    pxv = plsc.load_gather(px_v, [pidx], mask=lanemask)
                    pyv = plsc.load_gather(py_v, [pidx], mask=lanemask)
                    dxv = pxv - gxs
                    dyv = pyv - gys
                    d2v = dxv * dxv + dyv * dyv
                    d2v = jnp.where(lanemask, d2v, jnp.inf)
                    cm = (d2v < tsp) & (cc <= _CAP - 16)
                    cmi = cm.astype(i32)
                    pos = r * _CAP + cc + plsc.cumsum(cmi) - 1
                    plsc.store_scatter(cd2_v, [pos], d2v, mask=cm)
                    plsc.store_scatter(cdx_v, [pos], dxv, mask=cm)
                    plsc.store_scatter(cdy_v, [pos], dyv, mask=cm)
                    cc = cc + jnp.sum(cmi)
                return cc

            nv = (gcnt + 15) // 16
            lax.fori_loop(0, nv, p2, i32(0))

            vregs = [(cd2_v[pl.ds(r * _CAP + i * 16, 16)], iota + i * 16)
                     for i in range(_CAP // 16)]
            low = _sort128_low64(vregs)
            for i in range(4):
                sk, sv = low[i]
                dxs = plsc.load_gather(cdx_v, [sv + r * _CAP])
                dys = plsc.load_gather(cdy_v, [sv + r * _CAP])
                out_d2_v[pl.ds(r * _K + i * 16, 16)] = sk
                out_dx_v[pl.ds(r * _K + i * 16, 16)] = dxs
                out_dy_v[pl.ds(r * _K + i * 16, 16)] = dys
            return 0

        lax.fori_loop(0, 16, do_row, 0)

        obase = (wbase + base_l) * _K
        pltpu.sync_copy(out_d2_v, od2_hbm.at[pl.ds(obase, 16 * _K)])
        pltpu.sync_copy(out_dx_v, odx_hbm.at[pl.ds(obase, 16 * _K)])
        pltpu.sync_copy(out_dy_v, ody_hbm.at[pl.ds(obase, 16 * _K)])
        return 0

    lax.fori_loop(0, 8, do_chunk, 0)


def _mlp_body(rx_ref, ry_ref, d2_ref, w1_ref, b1_ref, w2_ref, b2_ref, out_ref,
              hbuf_ref):
    rx = rx_ref[...]
    ry = ry_ref[...]
    d2 = d2_ref[...]
    bm = rx.shape[0]
    dist = jnp.sqrt(jnp.maximum(d2, 1e-24))
    for s, (radius, k) in enumerate(_SCALES):
        w1x = w1_ref[2 * s : 2 * s + 1, :]
        w1y = w1_ref[2 * s + 1 : 2 * s + 2, :]
        b1 = b1_ref[s : s + 1, :]
        w2 = w2_ref[s * _HID : (s + 1) * _HID, :]
        b2 = b2_ref[s : s + 1, :]
        for j in range(k):
            hbuf_ref[j * bm : (j + 1) * bm, :] = jnp.maximum(
                rx[:, j : j + 1] * w1x + ry[:, j : j + 1] * w1y + b1, 0.0)
        f_all = jnp.dot(hbuf_ref[: k * bm, :], w2,
                        preferred_element_type=jnp.float32)
        acc = jnp.full(out_ref[:, : _OUT].shape, -jnp.inf, dtype=jnp.float32)
        anyv = jnp.zeros(acc[:, :1].shape, dtype=jnp.bool_)
        for j in range(k):
            valid = dist[:, j : j + 1] < radius
            acc = jnp.maximum(
                acc, jnp.where(valid, f_all[j * bm : (j + 1) * bm, :], -jnp.inf))
            anyv = jnp.logical_or(anyv, valid)
        pooled = jnp.where(anyv, acc + b2, 0.0)
        out_ref[:, s * _OUT : (s + 1) * _OUT] = pooled


def kernel(grid, points, W1_0, b1_0, W2_0, b2_0, W1_1, b1_1, W2_1, b2_1,
           W1_2, b1_2, W2_2, b2_2, chunk_size):
    gx = grid[:, 0:1]
    gy = grid[:, 1:2]
    px3 = points[:, 0].reshape(16, _NGRP)
    py3 = points[:, 1].reshape(16, _NGRP)

    bg = 512
    gmin, tau = pl.pallas_call(
        _gmin_tau_body,
        grid=(_G // bg,),
        in_specs=[
            pl.BlockSpec((bg, 1), lambda i: (i, 0)),
            pl.BlockSpec((bg, 1), lambda i: (i, 0)),
            pl.BlockSpec((16, _NGRP), lambda i: (0, 0)),
            pl.BlockSpec((16, _NGRP), lambda i: (0, 0)),
        ],
        out_specs=[
            pl.BlockSpec((bg, _NGRP), lambda i: (i, 0)),
            pl.BlockSpec((bg, 1), lambda i: (i, 0)),
        ],
        out_shape=[
            jax.ShapeDtypeStruct((_G, _NGRP), jnp.float32),
            jax.ShapeDtypeStruct((_G, 1), jnp.float32),
        ],
    )(gx, gy, px3, py3)

    mesh = plsc.VectorSubcoreMesh(core_axis_name="c", subcore_axis_name="s")
    sc = functools.partial(
        pl.kernel,
        mesh=mesh,
        compiler_params=pltpu.CompilerParams(needs_layout_passes=False),
        out_type=[
            jax.ShapeDtypeStruct((_G * _K,), jnp.float32),
            jax.ShapeDtypeStruct((_G * _K,), jnp.float32),
            jax.ShapeDtypeStruct((_G * _K,), jnp.float32),
        ],
        scratch_types=[
            pltpu.VMEM((_N,), jnp.float32),
            pltpu.VMEM((_N,), jnp.float32),
            pltpu.VMEM((128,), jnp.float32),
            pltpu.VMEM((128,), jnp.float32),
            pltpu.VMEM((128,), jnp.float32),
            pltpu.VMEM((16 * _NGRP,), jnp.float32),
            pltpu.VMEM((_CAP + 16,), jnp.int32),
            pltpu.VMEM((16 * _CAP,), jnp.float32),
            pltpu.VMEM((16 * _CAP,), jnp.float32),
            pltpu.VMEM((16 * _CAP,), jnp.float32),
            pltpu.VMEM((16 * _K,), jnp.float32),
            pltpu.VMEM((16 * _K,), jnp.float32),
            pltpu.VMEM((16 * _K,), jnp.float32),
        ],
    )(_sc_body)

    d2s, dxs, dys = sc(
        gmin.reshape(-1), tau.reshape(-1), gx.reshape(-1), gy.reshape(-1),
        points[:, 0], points[:, 1],
    )
    d2s = d2s.reshape(_G, _K)
    dxs = dxs.reshape(_G, _K)
    dys = dys.reshape(_G, _K)

    w1 = jnp.stack([W1_0[:, 0], W1_0[:, 1], W1_1[:, 0], W1_1[:, 1],
                    W1_2[:, 0], W1_2[:, 1]], axis=0)
    b1 = jnp.stack([b1_0, b1_1, b1_2], axis=0)
    w2 = jnp.concatenate([W2_0.T, W2_1.T, W2_2.T], axis=0)
    b2 = jnp.stack([b2_0, b2_1, b2_2], axis=0)

    bm = 256
    out = pl.pallas_call(
        _mlp_body,
        grid=(_G // bm,),
        in_specs=[
            pl.BlockSpec((bm, _K), lambda i: (i, 0)),
            pl.BlockSpec((bm, _K), lambda i: (i, 0)),
            pl.BlockSpec((bm, _K), lambda i: (i, 0)),
            pl.BlockSpec((6, _HID), lambda i: (0, 0)),
            pl.BlockSpec((3, _HID), lambda i: (0, 0)),
            pl.BlockSpec((3 * _HID, _OUT), lambda i: (0, 0)),
            pl.BlockSpec((3, _OUT), lambda i: (0, 0)),
        ],
        out_specs=pl.BlockSpec((bm, 3 * _OUT), lambda i: (i, 0)),
        out_shape=jax.ShapeDtypeStruct((_G, 3 * _OUT), jnp.float32),
        scratch_shapes=[pltpu.VMEM((_K * bm, _HID), jnp.float32)],
    )(dxs, dys, d2s, w1, b1, w2, b2)
    return out


# speedup vs baseline: 33.8489x; 1.0041x over previous
import functools

import jax
import jax.numpy as jnp
from jax import lax
from jax.experimental import pallas as pl
from jax.experimental.pallas import tpu as pltpu
from jax.experimental.pallas import tpu_sc as plsc

_SCALES = [(0.02, 16), (0.05, 32), (0.1, 64)]
_HID = 128
_OUT = 64
_G = 4096
_N = 16384
_K = 64
_NGRP = 1024
_CAP = 128


def _gmin_tau_body(gx_ref, gy_ref, px3_ref, py3_ref, gmin_ref, tau_ref):
    gx = gx_ref[...]
    gy = gy_ref[...]
    gmin = jnp.full((gx.shape[0], _NGRP), jnp.inf, dtype=jnp.float32)
    for s in range(16):
        dx = gx - px3_ref[s : s + 1, :]
        dy = gy - py3_ref[s : s + 1, :]
        gmin = jnp.minimum(gmin, dx * dx + dy * dy)
    gmin_ref[...] = gmin

    rmax2 = jnp.float32(0.01)

    def body(_, st):
        lo, clo, hi, chi = st
        denom = jnp.maximum(chi - clo, 1.0)
        frac = jnp.clip((64.0 - clo) / denom, 0.08, 0.92)
        mid = lo + (hi - lo) * frac
        c = jnp.sum((gmin < mid).astype(jnp.float32), axis=1, keepdims=True)
        p = c >= 64.0
        return (jnp.where(p, lo, mid), jnp.where(p, clo, c),
                jnp.where(p, mid, hi), jnp.where(p, c, chi))

    lo0 = jnp.zeros_like(gx)
    hi0 = jnp.full_like(gx, rmax2)
    chi0 = jnp.sum((gmin < rmax2).astype(jnp.float32), axis=1, keepdims=True)
    _, _, hi, _ = lax.fori_loop(0, 8, body, (lo0, jnp.zeros_like(gx), hi0, chi0))
    tau_ref[...] = hi * jnp.float32(1.0 + 2e-6)


def _ce(a, b):
    m = a[0] <= b[0]
    lo = (jnp.where(m, a[0], b[0]), jnp.where(m, a[1], b[1]))
    hi = (jnp.where(m, b[0], a[0]), jnp.where(m, b[1], a[1]))
    return lo, hi


def _vrev(a):
    return (lax.rev(a[0], (0,)), lax.rev(a[1], (0,)))


def _vsort(a):
    k, v = plsc.sort_key_val(a[0], a[1])
    return (k, v)


def _bmerge32(a, b):
    lo, hi = _ce(a, b)
    return _vsort(lo), _vsort(hi)


def _merge2(a, b):
    return _bmerge32(a, _vrev(b))


def _sort128_low64(vregs):
    v = [_vsort(x) for x in vregs]
    s32 = []
    for i in range(0, 8, 2):
        lo, hi = _merge2(v[i], v[i + 1])
        s32 += [lo, hi]
    s64 = []
    for i in range(0, 8, 4):
        a0, a1, b0, b1 = s32[i], s32[i + 1], s32[i + 2], s32[i + 3]
        w2, w3 = _vrev(b1), _vrev(b0)
        l0, h0 = _ce(a0, w2)
        l1, h1 = _ce(a1, w3)
        ll, lh = _bmerge32(l0, l1)
        hl, hh = _bmerge32(h0, h1)
        s64 += [ll, lh, hl, hh]
    a = s64[0:4]
    b = s64[4:8]
    w = [a[0], a[1], a[2], a[3], _vrev(b[3]), _vrev(b[2]), _vrev(b[1]), _vrev(b[0])]
    l = [_ce(w[i], w[i + 4])[0] for i in range(4)]
    p0, _h0 = _ce(l[0], l[2])
    p1, _h1 = _ce(l[1], l[3])
    q0, q1 = _bmerge32(p0, p1)
    r0, r1 = _bmerge32(_h0, _h1)
    return [q0, q1, r0, r1]


def _sc_body(gmin_hbm, tau_hbm, gx_hbm, gy_hbm, px_hbm, py_hbm,
             od2_hbm, odx_hbm, ody_hbm,
             px_v, py_v, tau_v, gx_v, gy_v, gmin_v, gid_v, cd2_v, cdx_v, cdy_v,
             out_d2_v, out_dx_v, out_dy_v):
    i32 = jnp.int32
    wid = lax.axis_index("s") * 2 + lax.axis_index("c")
    wbase = wid * 128

    pltpu.sync_copy(px_hbm, px_v)
    pltpu.sync_copy(py_hbm, py_v)
    pltpu.sync_copy(tau_hbm.at[pl.ds(wbase, 128)], tau_v)
    pltpu.sync_copy(gx_hbm.at[pl.ds(wbase, 128)], gx_v)
    pltpu.sync_copy(gy_hbm.at[pl.ds(wbase, 128)], gy_v)

    iota = lax.iota(i32, 16)
    inf16 = jnp.full((16,), jnp.inf, dtype=jnp.float32)

    def do_chunk(chunk, _):
        base_l = chunk * 16
        pltpu.sync_copy(gmin_hbm.at[pl.ds((wbase + base_l) * _NGRP, 16 * _NGRP)],
                        gmin_v)

        def do_row(r, _r):
            rl = base_l + r
            tsp = plsc.load_gather(tau_v, [jnp.full((16,), rl, i32)])
            gxs = plsc.load_gather(gx_v, [jnp.full((16,), rl, i32)])
            gys = plsc.load_gather(gy_v, [jnp.full((16,), rl, i32)])

            def p1(jv, gcntv):
                gv = gmin_v[pl.ds(r * _NGRP + jv * 16, 16)]
                m = gv < tsp
                mi = m.astype(i32)
                pos = gcntv + plsc.cumsum(mi) - 1
                plsc.store_scatter(gid_v, [pos], iota + jv * 16, mask=m)
                return gcntv + plsc.all_reduce_population_count(m)

            gcntv = lax.fori_loop(0, _NGRP // 16, p1, jnp.zeros((16,), i32))
            gcnt = jnp.max(gcntv)

            def pinit(w, _w):
                cd2_v[pl.ds(r * _CAP + w * 16, 16)] = inf16
                return 0

            lax.fori_loop(0, _CAP // 16, pinit, 0)

            def p2(b, cntv):
                gvreg = gid_v[pl.ds(b * 16, 16)]
                lanemask = (iota + b * 16) < gcntv
                cc = cntv
                for s in range(16):
                    pidx = gvreg + s * _NGRP
                    pxv = plsc.load_gather(px_v, [pidx], mask=lanemask)
                    pyv = plsc.load_gather(py_v, [pidx], mask=lanemask)
                    dxv = pxv - gxs
                    dyv = pyv - gys
                    d2v = dxv * dxv + dyv * dyv
                    d2v = jnp.where(lanemask, d2v, jnp.inf)
                    cm = (d2v < tsp) & (cc <= _CAP - 16)
                    cmi = cm.astype(i32)
                    pos = r * _CAP + cc + plsc.cumsum(cmi) - 1
                    plsc.store_scatter(cd2_v, [pos], d2v, mask=cm)
                    plsc.store_scatter(cdx_v, [pos], dxv, mask=cm)
                    plsc.store_scatter(cdy_v, [pos], dyv, mask=cm)
                    cc = cc + plsc.all_reduce_population_count(cm)
                return cc

            nv = (gcnt + 15) // 16
            lax.fori_loop(0, nv, p2, jnp.zeros((16,), i32))

            vregs = [(cd2_v[pl.ds(r * _CAP + i * 16, 16)], iota + i * 16)
                     for i in range(_CAP // 16)]
            low = _sort128_low64(vregs)
            for i in range(4):
                sk, sv = low[i]
                dxs = plsc.load_gather(cdx_v, [sv + r * _CAP])
                dys = plsc.load_gather(cdy_v, [sv + r * _CAP])
                out_d2_v[pl.ds(r * _K + i * 16, 16)] = sk
                out_dx_v[pl.ds(r * _K + i * 16, 16)] = dxs
                out_dy_v[pl.ds(r * _K + i * 16, 16)] = dys
            return 0

        lax.fori_loop(0, 16, do_row, 0)

        obase = (wbase + base_l) * _K
        pltpu.sync_copy(out_d2_v, od2_hbm.at[pl.ds(obase, 16 * _K)])
        pltpu.sync_copy(out_dx_v, odx_hbm.at[pl.ds(obase, 16 * _K)])
        pltpu.sync_copy(out_dy_v, ody_hbm.at[pl.ds(obase, 16 * _K)])
        return 0

    lax.fori_loop(0, 8, do_chunk, 0)


def _mlp_body(rx_ref, ry_ref, d2_ref, w1_ref, b1_ref, w2_ref, b2_ref, out_ref,
              hbuf_ref):
    rx = rx_ref[...]
    ry = ry_ref[...]
    d2 = d2_ref[...]
    bm = rx.shape[0]
    dist = jnp.sqrt(jnp.maximum(d2, 1e-24))
    for s, (radius, k) in enumerate(_SCALES):
        w1x = w1_ref[2 * s : 2 * s + 1, :]
        w1y = w1_ref[2 * s + 1 : 2 * s + 2, :]
        b1 = b1_ref[s : s + 1, :]
        w2 = w2_ref[s * _HID : (s + 1) * _HID, :]
        b2 = b2_ref[s : s + 1, :]
        for j in range(k):
            hbuf_ref[j * bm : (j + 1) * bm, :] = jnp.maximum(
                rx[:, j : j + 1] * w1x + ry[:, j : j + 1] * w1y + b1, 0.0)
        f_all = jnp.dot(hbuf_ref[: k * bm, :], w2,
                        preferred_element_type=jnp.float32)
        acc = jnp.full(out_ref[:, : _OUT].shape, -jnp.inf, dtype=jnp.float32)
        anyv = jnp.zeros(acc[:, :1].shape, dtype=jnp.bool_)
        for j in range(k):
            valid = dist[:, j : j + 1] < radius
            acc = jnp.maximum(
                acc, jnp.where(valid, f_all[j * bm : (j + 1) * bm, :], -jnp.inf))
            anyv = jnp.logical_or(anyv, valid)
        pooled = jnp.where(anyv, acc + b2, 0.0)
        out_ref[:, s * _OUT : (s + 1) * _OUT] = pooled


def kernel(grid, points, W1_0, b1_0, W2_0, b2_0, W1_1, b1_1, W2_1, b2_1,
           W1_2, b1_2, W2_2, b2_2, chunk_size):
    gx = grid[:, 0:1]
    gy = grid[:, 1:2]
    px3 = points[:, 0].reshape(16, _NGRP)
    py3 = points[:, 1].reshape(16, _NGRP)

    bg = 512
    gmin, tau = pl.pallas_call(
        _gmin_tau_body,
        grid=(_G // bg,),
        in_specs=[
            pl.BlockSpec((bg, 1), lambda i: (i, 0)),
            pl.BlockSpec((bg, 1), lambda i: (i, 0)),
            pl.BlockSpec((16, _NGRP), lambda i: (0, 0)),
            pl.BlockSpec((16, _NGRP), lambda i: (0, 0)),
        ],
        out_specs=[
            pl.BlockSpec((bg, _NGRP), lambda i: (i, 0)),
            pl.BlockSpec((bg, 1), lambda i: (i, 0)),
        ],
        out_shape=[
            jax.ShapeDtypeStruct((_G, _NGRP), jnp.float32),
            jax.ShapeDtypeStruct((_G, 1), jnp.float32),
        ],
    )(gx, gy, px3, py3)

    mesh = plsc.VectorSubcoreMesh(core_axis_name="c", subcore_axis_name="s")
    sc = functools.partial(
        pl.kernel,
        mesh=mesh,
        compiler_params=pltpu.CompilerParams(needs_layout_passes=False),
        out_type=[
            jax.ShapeDtypeStruct((_G * _K,), jnp.float32),
            jax.ShapeDtypeStruct((_G * _K,), jnp.float32),
            jax.ShapeDtypeStruct((_G * _K,), jnp.float32),
        ],
        scratch_types=[
            pltpu.VMEM((_N,), jnp.float32),
            pltpu.VMEM((_N,), jnp.float32),
            pltpu.VMEM((128,), jnp.float32),
            pltpu.VMEM((128,), jnp.float32),
            pltpu.VMEM((128,), jnp.float32),
            pltpu.VMEM((16 * _NGRP,), jnp.float32),
            pltpu.VMEM((_CAP + 16,), jnp.int32),
            pltpu.VMEM((16 * _CAP,), jnp.float32),
            pltpu.VMEM((16 * _CAP,), jnp.float32),
            pltpu.VMEM((16 * _CAP,), jnp.float32),
            pltpu.VMEM((16 * _K,), jnp.float32),
            pltpu.VMEM((16 * _K,), jnp.float32),
            pltpu.VMEM((16 * _K,), jnp.float32),
        ],
    )(_sc_body)

    d2s, dxs, dys = sc(
        gmin.reshape(-1), tau.reshape(-1), gx.reshape(-1), gy.reshape(-1),
        points[:, 0], points[:, 1],
    )
    d2s = d2s.reshape(_G, _K)
    dxs = dxs.reshape(_G, _K)
    dys = dys.reshape(_G, _K)

    w1 = jnp.stack([W1_0[:, 0], W1_0[:, 1], W1_1[:, 0], W1_1[:, 1],
                    W1_2[:, 0], W1_2[:, 1]], axis=0)
    b1 = jnp.stack([b1_0, b1_1, b1_2], axis=0)
    w2 = jnp.concatenate([W2_0.T, W2_1.T, W2_2.T], axis=0)
    b2 = jnp.stack([b2_0, b2_1, b2_2], axis=0)

    bm = 256
    out = pl.pallas_call(
        _mlp_body,
        grid=(_G // bm,),
        in_specs=[
            pl.BlockSpec((bm, _K), lambda i: (i, 0)),
            pl.BlockSpec((bm, _K), lambda i: (i, 0)),
            pl.BlockSpec((bm, _K), lambda i: (i, 0)),
            pl.BlockSpec((6, _HID), lambda i: (0, 0)),
            pl.BlockSpec((3, _HID), lambda i: (0, 0)),
            pl.BlockSpec((3 * _HID, _OUT), lambda i: (0, 0)),
            pl.BlockSpec((3, _OUT), lambda i: (0, 0)),
        ],
        out_specs=pl.BlockSpec((bm, 3 * _OUT), lambda i: (i, 0)),
        out_shape=jax.ShapeDtypeStruct((_G, 3 * _OUT), jnp.float32),
        scratch_shapes=[pltpu.VMEM((_K * bm, _HID), jnp.float32)],
    )(dxs, dys, d2s, w1, b1, w2, b2)
    return out


# speedup vs baseline: 33.8599x; 1.0003x over previous
import functools

import jax
import jax.numpy as jnp
from jax import lax
from jax.experimental import pallas as pl
from jax.experimental.pallas import tpu as pltpu
from jax.experimental.pallas import tpu_sc as plsc

_SCALES = [(0.02, 16), (0.05, 32), (0.1, 64)]
_HID = 128
_OUT = 64
_G = 4096
_N = 16384
_K = 64
_NGRP = 1024
_CAP = 128


def _gmin_tau_body(gx_ref, gy_ref, px3_ref, py3_ref, gmin_ref, tau_ref):
    gx = gx_ref[...]
    gy = gy_ref[...]
    gmin = jnp.full((gx.shape[0], _NGRP), jnp.inf, dtype=jnp.float32)
    for s in range(16):
        dx = gx - px3_ref[s : s + 1, :]
        dy = gy - py3_ref[s : s + 1, :]
        gmin = jnp.minimum(gmin, dx * dx + dy * dy)
    gmin_ref[...] = gmin

    rmax2 = jnp.float32(0.01)

    def body(_, st):
        lo, clo, hi, chi = st
        denom = jnp.maximum(chi - clo, 1.0)
        frac = jnp.clip((64.0 - clo) / denom, 0.08, 0.92)
        mid = lo + (hi - lo) * frac
        c = jnp.sum((gmin < mid).astype(jnp.float32), axis=1, keepdims=True)
        p = c >= 64.0
        return (jnp.where(p, lo, mid), jnp.where(p, clo, c),
                jnp.where(p, mid, hi), jnp.where(p, c, chi))

    lo0 = jnp.zeros_like(gx)
    hi0 = jnp.full_like(gx, rmax2)
    chi0 = jnp.sum((gmin < rmax2).astype(jnp.float32), axis=1, keepdims=True)
    _, _, hi, _ = lax.fori_loop(0, 8, body, (lo0, jnp.zeros_like(gx), hi0, chi0))
    tau_ref[...] = hi * jnp.float32(1.0 + 2e-6)


def _ce(a, b):
    m = a[0] <= b[0]
    lo = (jnp.where(m, a[0], b[0]), jnp.where(m, a[1], b[1]))
    hi = (jnp.where(m, b[0], a[0]), jnp.where(m, b[1], a[1]))
    return lo, hi


def _vrev(a):
    return (lax.rev(a[0], (0,)), lax.rev(a[1], (0,)))


def _vsort(a):
    k, v = plsc.sort_key_val(a[0], a[1])
    return (k, v)


def _bmerge32(a, b):
    lo, hi = _ce(a, b)
    return _vsort(lo), _vsort(hi)


def _merge2(a, b):
    return _bmerge32(a, _vrev(b))


def _sort128_low64(vregs):
    v = [_vsort(x) for x in vregs]
    s32 = []
    for i in range(0, 8, 2):
        lo, hi = _merge2(v[i], v[i + 1])
        s32 += [lo, hi]
    s64 = []
    for i in range(0, 8, 4):
        a0, a1, b0, b1 = s32[i], s32[i + 1], s32[i + 2], s32[i + 3]
        w2, w3 = _vrev(b1), _vrev(b0)
        l0, h0 = _ce(a0, w2)
        l1, h1 = _ce(a1, w3)
        ll, lh = _bmerge32(l0, l1)
        hl, hh = _bmerge32(h0, h1)
        s64 += [ll, lh, hl, hh]
    a = s64[0:4]
    b = s64[4:8]
    w = [a[0], a[1], a[2], a[3], _vrev(b[3]), _vrev(b[2]), _vrev(b[1]), _vrev(b[0])]
    l = [_ce(w[i], w[i + 4])[0] for i in range(4)]
    p0, _h0 = _ce(l[0], l[2])
    p1, _h1 = _ce(l[1], l[3])
    q0, q1 = _bmerge32(p0, p1)
    r0, r1 = _bmerge32(_h0, _h1)
    return [q0, q1, r0, r1]


def _sc_body(gmin_hbm, tau_hbm, gx_hbm, gy_hbm, px_hbm, py_hbm,
             od2_hbm, odx_hbm, ody_hbm,
             px_v, py_v, tau_v, gx_v, gy_v, gmin_v, gid_v, cd2_v, cdx_v, cdy_v,
             out_d2_v, out_dx_v, out_dy_v):
    i32 = jnp.int32
    wid = lax.axis_index("s") * 2 + lax.axis_index("c")
    wbase = wid * 128

    pltpu.sync_copy(px_hbm, px_v)
    pltpu.sync_copy(py_hbm, py_v)
    pltpu.sync_copy(tau_hbm.at[pl.ds(wbase, 128)], tau_v)
    pltpu.sync_copy(gx_hbm.at[pl.ds(wbase, 128)], gx_v)
    pltpu.sync_copy(gy_hbm.at[pl.ds(wbase, 128)], gy_v)

    iota = lax.iota(i32, 16)
    inf16 = jnp.full((16,), jnp.inf, dtype=jnp.float32)

    def do_chunk(chunk, _):
        base_l = chunk * 16
        pltpu.sync_copy(gmin_hbm.at[pl.ds((wbase + base_l) * _NGRP, 16 * _NGRP)],
                        gmin_v)

        def do_row(r, _r):
            rl = base_l + r
            tsp = plsc.load_gather(tau_v, [jnp.full((16,), rl, i32)])
            gxs = plsc.load_gather(gx_v, [jnp.full((16,), rl, i32)])
            gys = plsc.load_gather(gy_v, [jnp.full((16,), rl, i32)])

            def p1(q, gcntv):
                acc = gcntv
                for u in range(4):
                    jv = q * 4 + u
                    gv = gmin_v[pl.ds(r * _NGRP + jv * 16, 16)]
                    m = gv < tsp
                    mi = m.astype(i32)
                    pos = acc + plsc.cumsum(mi) - 1
                    plsc.store_scatter(gid_v, [pos], iota + jv * 16, mask=m)
                    acc = acc + plsc.all_reduce_population_count(m)
                return acc

            gcntv = lax.fori_loop(0, _NGRP // 64, p1, jnp.zeros((16,), i32))
            gcnt = jnp.max(gcntv)

            def pinit(w, _w):
                cd2_v[pl.ds(r * _CAP + w * 16, 16)] = inf16
                return 0

            lax.fori_loop(0, _CAP // 16, pinit, 0)

            def p2(b, cntv):
                gvreg = gid_v[pl.ds(b * 16, 16)]
                lanemask = (iota + b * 16) < gcntv
                cc = cntv
                for s in range(16):
                    pidx = gvreg + s * _NGRP
                    pxv = plsc.load_gather(px_v, [pidx], mask=lanemask)
                    pyv = plsc.load_gather(py_v, [pidx], mask=lanemask)
                    dxv = pxv - gxs
                    dyv = pyv - gys
                    d2v = dxv * dxv + dyv * dyv
                    d2v = jnp.where(lanemask, d2v, jnp.inf)
                    cm = (d2v < tsp) & (cc <= _CAP - 16)
                    cmi = cm.astype(i32)
                    pos = r * _CAP + cc + plsc.cumsum(cmi) - 1
                    plsc.store_scatter(cd2_v, [pos], d2v, mask=cm)
                    plsc.store_scatter(cdx_v, [pos], dxv, mask=cm)
                    plsc.store_scatter(cdy_v, [pos], dyv, mask=cm)
                    cc = cc + plsc.all_reduce_population_count(cm)
                return cc

            nv = (gcnt + 15) // 16
            lax.fori_loop(0, nv, p2, jnp.zeros((16,), i32))

            vregs = [(cd2_v[pl.ds(r * _CAP + i * 16, 16)], iota + i * 16)
                     for i in range(_CAP // 16)]
            low = _sort128_low64(vregs)
            for i in range(4):
                sk, sv = low[i]
                dxs = plsc.load_gather(cdx_v, [sv + r * _CAP])
                dys = plsc.load_gather(cdy_v, [sv + r * _CAP])
                out_d2_v[pl.ds(r * _K + i * 16, 16)] = sk
                out_dx_v[pl.ds(r * _K + i * 16, 16)] = dxs
                out_dy_v[pl.ds(r * _K + i * 16, 16)] = dys
            return 0

        lax.fori_loop(0, 16, do_row, 0)

        obase = (wbase + base_l) * _K
        pltpu.sync_copy(out_d2_v, od2_hbm.at[pl.ds(obase, 16 * _K)])
        pltpu.sync_copy(out_dx_v, odx_hbm.at[pl.ds(obase, 16 * _K)])
        pltpu.sync_copy(out_dy_v, ody_hbm.at[pl.ds(obase, 16 * _K)])
        return 0

    lax.fori_loop(0, 8, do_chunk, 0)


def _mlp_body(rx_ref, ry_ref, d2_ref, w1_ref, b1_ref, w2_ref, b2_ref, out_ref,
              hbuf_ref):
    rx = rx_ref[...]
    ry = ry_ref[...]
    d2 = d2_ref[...]
    bm = rx.shape[0]
    dist = jnp.sqrt(jnp.maximum(d2, 1e-24))
    for s, (radius, k) in enumerate(_SCALES):
        w1x = w1_ref[2 * s : 2 * s + 1, :]
        w1y = w1_ref[2 * s + 1 : 2 * s + 2, :]
        b1 = b1_ref[s : s + 1, :]
        w2 = w2_ref[s * _HID : (s + 1) * _HID, :]
        b2 = b2_ref[s : s + 1, :]
        for j in range(k):
            hbuf_ref[j * bm : (j + 1) * bm, :] = jnp.maximum(
                rx[:, j : j + 1] * w1x + ry[:, j : j + 1] * w1y + b1, 0.0)
        f_all = jnp.dot(hbuf_ref[: k * bm, :], w2,
                        preferred_element_type=jnp.float32)
        acc = jnp.full(out_ref[:, : _OUT].shape, -jnp.inf, dtype=jnp.float32)
        anyv = jnp.zeros(acc[:, :1].shape, dtype=jnp.bool_)
        for j in range(k):
            valid = dist[:, j : j + 1] < radius
            acc = jnp.maximum(
                acc, jnp.where(valid, f_all[j * bm : (j + 1) * bm, :], -jnp.inf))
            anyv = jnp.logical_or(anyv, valid)
        pooled = jnp.where(anyv, acc + b2, 0.0)
        out_ref[:, s * _OUT : (s + 1) * _OUT] = pooled


def kernel(grid, points, W1_0, b1_0, W2_0, b2_0, W1_1, b1_1, W2_1, b2_1,
           W1_2, b1_2, W2_2, b2_2, chunk_size):
    gx = grid[:, 0:1]
    gy = grid[:, 1:2]
    px3 = points[:, 0].reshape(16, _NGRP)
    py3 = points[:, 1].reshape(16, _NGRP)

    bg = 512
    gmin, tau = pl.pallas_call(
        _gmin_tau_body,
        grid=(_G // bg,),
        in_specs=[
            pl.BlockSpec((bg, 1), lambda i: (i, 0)),
            pl.BlockSpec((bg, 1), lambda i: (i, 0)),
            pl.BlockSpec((16, _NGRP), lambda i: (0, 0)),
            pl.BlockSpec((16, _NGRP), lambda i: (0, 0)),
        ],
        out_specs=[
            pl.BlockSpec((bg, _NGRP), lambda i: (i, 0)),
            pl.BlockSpec((bg, 1), lambda i: (i, 0)),
        ],
        out_shape=[
            jax.ShapeDtypeStruct((_G, _NGRP), jnp.float32),
            jax.ShapeDtypeStruct((_G, 1), jnp.float32),
        ],
    )(gx, gy, px3, py3)

    mesh = plsc.VectorSubcoreMesh(core_axis_name="c", subcore_axis_name="s")
    sc = functools.partial(
        pl.kernel,
        mesh=mesh,
        compiler_params=pltpu.CompilerParams(needs_layout_passes=False),
        out_type=[
            jax.ShapeDtypeStruct((_G * _K,), jnp.float32),
            jax.ShapeDtypeStruct((_G * _K,), jnp.float32),
            jax.ShapeDtypeStruct((_G * _K,), jnp.float32),
        ],
        scratch_types=[
            pltpu.VMEM((_N,), jnp.float32),
            pltpu.VMEM((_N,), jnp.float32),
            pltpu.VMEM((128,), jnp.float32),
            pltpu.VMEM((128,), jnp.float32),
            pltpu.VMEM((128,), jnp.float32),
            pltpu.VMEM((16 * _NGRP,), jnp.float32),
            pltpu.VMEM((_CAP + 16,), jnp.int32),
            pltpu.VMEM((16 * _CAP,), jnp.float32),
            pltpu.VMEM((16 * _CAP,), jnp.float32),
            pltpu.VMEM((16 * _CAP,), jnp.float32),
            pltpu.VMEM((16 * _K,), jnp.float32),
            pltpu.VMEM((16 * _K,), jnp.float32),
            pltpu.VMEM((16 * _K,), jnp.float32),
        ],
    )(_sc_body)

    d2s, dxs, dys = sc(
        gmin.reshape(-1), tau.reshape(-1), gx.reshape(-1), gy.reshape(-1),
        points[:, 0], points[:, 1],
    )
    d2s = d2s.reshape(_G, _K)
    dxs = dxs.reshape(_G, _K)
    dys = dys.reshape(_G, _K)

    w1 = jnp.stack([W1_0[:, 0], W1_0[:, 1], W1_1[:, 0], W1_1[:, 1],
                    W1_2[:, 0], W1_2[:, 1]], axis=0)
    b1 = jnp.stack([b1_0, b1_1, b1_2], axis=0)
    w2 = jnp.concatenate([W2_0.T, W2_1.T, W2_2.T], axis=0)
    b2 = jnp.stack([b2_0, b2_1, b2_2], axis=0)

    bm = 256
    out = pl.pallas_call(
        _mlp_body,
        grid=(_G // bm,),
        in_specs=[
            pl.BlockSpec((bm, _K), lambda i: (i, 0)),
            pl.BlockSpec((bm, _K), lambda i: (i, 0)),
            pl.BlockSpec((bm, _K), lambda i: (i, 0)),
            pl.BlockSpec((6, _HID), lambda i: (0, 0)),
            pl.BlockSpec((3, _HID), lambda i: (0, 0)),
            pl.BlockSpec((3 * _HID, _OUT), lambda i: (0, 0)),
            pl.BlockSpec((3, _OUT), lambda i: (0, 0)),
        ],
        out_specs=pl.BlockSpec((bm, 3 * _OUT), lambda i: (i, 0)),
        out_shape=jax.ShapeDtypeStruct((_G, 3 * _OUT), jnp.float32),
        scratch_shapes=[pltpu.VMEM((_K * bm, _HID), jnp.float32)],
    )(dxs, dys, d2s, w1, b1, w2, b2)
    return out
